# baseline (speedup 1.0000x reference)
import functools
from math import sqrt

import jax
import jax.numpy as jnp
from jax import lax
from jax.experimental import pallas as pl
from jax.experimental.pallas import tpu as pltpu
from jax.experimental.pallas import tpu_sc as plsc

N = 10000
E = 320000
F = 100000
ENC = 128
DEC = 128
CH = 8
SH = 8
HD = 16
NEG_SLOPE = 0.1
HFFP = 384

ROW_BLK = 1000

NC, NS = 2, 16
NP = 10240
NW = NC * NS
EC = 128
NCHUNK = E // EC
KMAX = (NCHUNK + NS - 1) // NS
TR = NP // NS
HW = 4
VW = HW * HD
AW = VW + 16


def _swish(x):
    return x * jax.nn.sigmoid(x)


def _rms_norm(x, w):
    return x / jnp.sqrt(jnp.mean(x * x, axis=-1, keepdims=True) + 1e-6) * w


def _pre1_body(fb_ref, root_ref, wa_ref, wc_ref, src_t_ref, xp_t_ref):
    fb = fb_ref[...]
    root = root_ref[...]
    ctxp = jnp.dot(fb, wc_ref[...], preferred_element_type=jnp.float32)
    cp = jnp.dot(fb, wa_ref[...][:DEC], preferred_element_type=jnp.float32)
    xp = jnp.dot(root, wa_ref[...][DEC:], preferred_element_type=jnp.float32)
    z12 = jnp.zeros((fb.shape[0], 12), jnp.float32)
    for c in range(NC):
        src_t_ref[c] = jnp.concatenate(
            [ctxp[:, c * VW:(c + 1) * VW], cp[:, c * HW:(c + 1) * HW], z12],
            axis=1)
        xp_t_ref[c] = jnp.concatenate(
            [xp[:, c * HW:(c + 1) * HW], z12], axis=1)


def _pre1(fb, root, w_attn, w_ctx2x):
    grid = (N // ROW_BLK,)
    return pl.pallas_call(
        _pre1_body,
        grid=grid,
        in_specs=[
            pl.BlockSpec((ROW_BLK, DEC), lambda i: (i, 0)),
            pl.BlockSpec((ROW_BLK, ENC), lambda i: (i, 0)),
            pl.BlockSpec((DEC + ENC, CH), lambda i: (0, 0)),
            pl.BlockSpec((DEC, ENC), lambda i: (0, 0)),
        ],
        out_specs=[
            pl.BlockSpec((NC, ROW_BLK, AW), lambda i: (0, i, 0)),
            pl.BlockSpec((NC, ROW_BLK, 16), lambda i: (0, i, 0)),
        ],
        out_shape=[
            jax.ShapeDtypeStruct((NC, N, AW), jnp.float32),
            jax.ShapeDtypeStruct((NC, N, 16), jnp.float32),
        ],
    )(fb, root, w_attn, w_ctx2x)


def _combine(acc_ref):
    unm = jnp.concatenate([acc_ref[0][:, :VW], acc_ref[1][:, :VW]], axis=1)
    s = jnp.concatenate(
        [acc_ref[0][:, VW:VW + HW], acc_ref[1][:, VW:VW + HW]], axis=1)
    has_in = acc_ref[0][:, VW + HW:VW + HW + 1] > 0
    return unm, s, has_in


def _mid_body(acc_ref, root_ref, wx2g_ref, wrms_ref, wq_ref, wkv_ref, p_ref,
              root1_ref, q2_ref, kv2_ref):
    root = root_ref[...]
    p = p_ref[...]
    unm, s, has_in = _combine(acc_ref)
    inv = 1.0 / (s + 1e-16)
    mess = unm * jnp.dot(inv, p, preferred_element_type=jnp.float32)
    gates = jnp.dot(
        jnp.dot(root, wx2g_ref[...], preferred_element_type=jnp.float32),
        p, preferred_element_type=jnp.float32)
    gates = jnp.where(jnp.broadcast_to(has_in, gates.shape), gates, 1.0)
    out = gates * root + (1.0 - gates) * mess
    root1 = _rms_norm(root + out, wrms_ref[...][0])
    root1_ref[...] = root1
    q = jnp.dot(root1, wq_ref[...], preferred_element_type=jnp.float32)
    kv = jnp.dot(root1, wkv_ref[...], preferred_element_type=jnp.float32)
    for c in range(NC):
        q2_ref[c] = q[:, c * VW:(c + 1) * VW]
        kv2_ref[c] = jnp.concatenate(
            [kv[:, c * VW:(c + 1) * VW],
             kv[:, ENC + c * VW:ENC + (c + 1) * VW]], axis=1)


def _mid(acc, root, w_x2g, rms_node_w, wq, wkv, p):
    grid = (N // ROW_BLK,)
    return pl.pallas_call(
        _mid_body,
        grid=grid,
        in_specs=[
            pl.BlockSpec((NC, ROW_BLK, AW), lambda i: (0, i, 0)),
            pl.BlockSpec((ROW_BLK, ENC), lambda i: (i, 0)),
            pl.BlockSpec((ENC, CH), lambda i: (0, 0)),
            pl.BlockSpec((1, ENC), lambda i: (0, 0)),
            pl.BlockSpec((ENC, ENC), lambda i: (0, 0)),
            pl.BlockSpec((ENC, 2 * ENC), lambda i: (0, 0)),
            pl.BlockSpec((CH, ENC), lambda i: (0, 0)),
        ],
        out_specs=[
            pl.BlockSpec((ROW_BLK, ENC), lambda i: (i, 0)),
            pl.BlockSpec((NC, ROW_BLK, VW), lambda i: (0, i, 0)),
            pl.BlockSpec((NC, ROW_BLK, 2 * VW), lambda i: (0, i, 0)),
        ],
        out_shape=[
            jax.ShapeDtypeStruct((N, ENC), jnp.float32),
            jax.ShapeDtypeStruct((NC, N, VW), jnp.float32),
            jax.ShapeDtypeStruct((NC, N, 2 * VW), jnp.float32),
        ],
    )(acc, root, w_x2g, rms_node_w.reshape(1, ENC), wq, wkv, p)


def _post_body(acc_ref, root1_ref, wr_ref, wg_ref, wu_ref, wd_ref, wf_ref,
               p_ref, root3_ref):
    root1 = root1_ref[...]
    unm, s, _ = _combine(acc_ref)
    inv = 1.0 / (s + 1e-16)
    mha = unm * jnp.dot(inv, p_ref[...], preferred_element_type=jnp.float32)
    root2 = _rms_norm(root1 + mha, wr_ref[...][0])
    gate = jnp.dot(root2, wg_ref[...], preferred_element_type=jnp.float32)
    up = jnp.dot(root2, wu_ref[...], preferred_element_type=jnp.float32)
    ffn = jnp.dot(_swish(gate) * up, wd_ref[...], preferred_element_type=jnp.float32)
    root3_ref[...] = _rms_norm(root2 + ffn, wf_ref[...][0])


def _post(acc, root1, rms_root_w, wg, wu, wd, rms_ffn_w, p):
    grid = (N // ROW_BLK,)
    return pl.pallas_call(
        _post_body,
        grid=grid,
        in_specs=[
            pl.BlockSpec((NC, ROW_BLK, AW), lambda i: (0, i, 0)),
            pl.BlockSpec((ROW_BLK, ENC), lambda i: (i, 0)),
            pl.BlockSpec((1, ENC), lambda i: (0, 0)),
            pl.BlockSpec((ENC, HFFP), lambda i: (0, 0)),
            pl.BlockSpec((ENC, HFFP), lambda i: (0, 0)),
            pl.BlockSpec((HFFP, ENC), lambda i: (0, 0)),
            pl.BlockSpec((1, ENC), lambda i: (0, 0)),
            pl.BlockSpec((CH, ENC), lambda i: (0, 0)),
        ],
        out_specs=pl.BlockSpec((ROW_BLK, ENC), lambda i: (i, 0)),
        out_shape=jax.ShapeDtypeStruct((N, ENC), jnp.float32),
    )(acc, root1, rms_root_w.reshape(1, ENC), wg, wu, wd,
      rms_ffn_w.reshape(1, ENC), p)


FR_BLK = 1000


def _fringe_body(fr_ref, gr_ref, wfg_ref, bfg_ref, wrf_ref, brf_ref, out_ref):
    fr = fr_ref[...]
    fg = _swish(jnp.dot(fr, wfg_ref[...], preferred_element_type=jnp.float32)
                + bfg_ref[...][0])
    r2f = jnp.dot(gr_ref[...], wrf_ref[...], preferred_element_type=jnp.float32) \
        + brf_ref[...][0]
    out_ref[...] = r2f * fg


def _fringe(fr, gathered, w_fgate, b_fgate, w_r2f, b_r2f):
    grid = (F // FR_BLK,)
    return pl.pallas_call(
        _fringe_body,
        grid=grid,
        in_specs=[
            pl.BlockSpec((FR_BLK, DEC), lambda i: (i, 0)),
            pl.BlockSpec((FR_BLK, ENC), lambda i: (i, 0)),
            pl.BlockSpec((DEC, DEC), lambda i: (0, 0)),
            pl.BlockSpec((1, DEC), lambda i: (0, 0)),
            pl.BlockSpec((ENC, DEC), lambda i: (0, 0)),
            pl.BlockSpec((1, DEC), lambda i: (0, 0)),
        ],
        out_specs=pl.BlockSpec((FR_BLK, DEC), lambda i: (i, 0)),
        out_shape=jax.ShapeDtypeStruct((F, DEC), jnp.float32),
    )(fr, gathered, w_fgate, b_fgate.reshape(1, DEC), w_r2f,
      b_r2f.reshape(1, DEC))


_SC_PARAMS = pltpu.CompilerParams(use_tc_tiling_on_sc=False,
                                  needs_layout_passes=False)
_MESH = dict(core_axis_name="c", subcore_axis_name="s", num_cores=NC,
             num_subcores=NS)


def _splat16(v, i):
    idx = jnp.full((16, 1), i, jnp.int32)
    return lax.gather(
        v, idx,
        lax.GatherDimensionNumbers(offset_dims=(), collapsed_slice_dims=(0,),
                                   start_index_map=(0,)),
        (1,), mode=lax.GatherScatterMode.PROMISE_IN_BOUNDS)


def _adjust(dst_idx_ref, src_idx_ref, off):
    for j in range(EC // 16):
        sl = pl.ds(j * 16, 16)
        dst_idx_ref[sl] = src_idx_ref[sl] + off


def _acc_writeback(acc_sh, out_hbm, cid, sid):
    plsc.subcore_barrier()
    base = sid * TR
    pltpu.sync_copy(acc_sh.at[pl.ds(base, TR)],
                    out_hbm.at[pl.ds(cid * NP + base, TR)])


def _acc_zero(zeros_hbm, acc_sh, sid):
    base = sid * TR
    pltpu.sync_copy(zeros_hbm.at[pl.ds(base, TR)], acc_sh.at[pl.ds(base, TR)])
    plsc.subcore_barrier()


def _edge1_sc(src, dst, src2, xp2, zeros):
    mesh = plsc.VectorSubcoreMesh(**_MESH)

    @functools.partial(
        pl.kernel,
        out_type=jax.ShapeDtypeStruct((NC * NP, AW), jnp.float32),
        mesh=mesh,
        compiler_params=_SC_PARAMS,
        scratch_types=[pltpu.VMEM_SHARED((NP, AW), jnp.float32)] + 2 * [
            pltpu.VMEM((EC,), jnp.int32),
            pltpu.VMEM((EC,), jnp.int32),
            pltpu.VMEM((EC,), jnp.int32),
            pltpu.VMEM((EC, AW), jnp.float32),
            pltpu.VMEM((EC, 16), jnp.float32),
            pltpu.SemaphoreType.DMA,
            pltpu.SemaphoreType.DMA,
            pltpu.SemaphoreType.DMA,
        ],
    )
    def k(src_hbm, dst_hbm, st_hbm, xp_hbm, z_hbm, out_hbm, acc_sh, *bufs):
        buf_a, buf_b = bufs[:8], bufs[8:]
        cid = lax.axis_index("c")
        sid = lax.axis_index("s")
        _acc_zero(z_hbm, acc_sh, sid)
        off = cid * N

        def issue(ci, bufset, drain):
            sidx, didx, didx2, rows, xpr, sem1, sem2, semsc = bufset
            if drain is not None:
                @pl.when(drain)
                def _():
                    pltpu.make_async_copy(rows, acc_sh.at[didx], semsc).wait()
            eb = ci * EC
            pltpu.sync_copy(src_hbm.at[pl.ds(eb, EC)], sidx)
            pltpu.sync_copy(dst_hbm.at[pl.ds(eb, EC)], didx)
            _adjust(sidx, sidx, off)
            _adjust(didx2, didx, off)
            pltpu.async_copy(st_hbm.at[sidx], rows, sem1)
            pltpu.async_copy(xp_hbm.at[didx2], xpr, sem2)

        def process(bufset):
            sidx, didx, didx2, rows, xpr, sem1, sem2, semsc = bufset
            pltpu.make_async_copy(st_hbm.at[sidx], rows, sem1).wait()
            pltpu.make_async_copy(xp_hbm.at[didx2], xpr, sem2).wait()

            @plsc.parallel_loop(0, EC, unroll=16)
            def ebody(e):
                cp = rows[e, pl.ds(VW, 16)]
                z = cp + xpr[e, :]
                a = jnp.maximum(z, NEG_SLOPE * z)
                w = jnp.exp(a)
                rows[e, pl.ds(VW, 16)] = w
                for h in range(HW):
                    wh = _splat16(w, h)
                    rows[e, pl.ds(h * HD, HD)] = wh * rows[e, pl.ds(h * HD, HD)]

            pltpu.async_copy(rows, acc_sh.at[didx], semsc, add=True)

        issue(sid, buf_a, None)

        def guarded(k_, _):
            ci = sid + k_ * NS
            nxt = ci + NS

            @pl.when(ci < NCHUNK)
            def _():
                @pl.when(k_ % 2 == 0)
                def _():
                    @pl.when(nxt < NCHUNK)
                    def _():
                        issue(nxt, buf_b, k_ >= 1)
                    process(buf_a)

                @pl.when(k_ % 2 == 1)
                def _():
                    @pl.when(nxt < NCHUNK)
                    def _():
                        issue(nxt, buf_a, k_ >= 1)
                    process(buf_b)
            return 0

        lax.fori_loop(0, KMAX, guarded, 0)
        for bset in (buf_a, buf_b):
            pltpu.make_async_copy(bset[3], acc_sh.at[bset[1]], bset[7]).wait()
        _acc_writeback(acc_sh, out_hbm, cid, sid)

    return k(src, dst, src2, xp2, zeros).reshape(NC, NP, AW)


def _edge2_sc(s2, d2, q2, kv2, attr, zeros):
    mesh = plsc.VectorSubcoreMesh(**_MESH)

    @functools.partial(
        pl.kernel,
        out_type=jax.ShapeDtypeStruct((NC * NP, AW), jnp.float32),
        mesh=mesh,
        compiler_params=_SC_PARAMS,
        scratch_types=[
            pltpu.VMEM_SHARED((NP, AW), jnp.float32),
        ] + 2 * [
            pltpu.VMEM((EC,), jnp.int32),
            pltpu.VMEM((EC,), jnp.int32),
            pltpu.VMEM((EC,), jnp.int32),
            pltpu.VMEM((EC, VW), jnp.float32),
            pltpu.VMEM((EC, 2 * VW), jnp.float32),
            pltpu.VMEM((EC, 16), jnp.float32),
            pltpu.VMEM((EC, AW), jnp.float32),
            pltpu.SemaphoreType.DMA,
            pltpu.SemaphoreType.DMA,
            pltpu.SemaphoreType.DMA,
            pltpu.SemaphoreType.DMA,
        ],
    )
    def k(s2_hbm, d2_hbm, q_hbm, kv_hbm, at_hbm, z_hbm, out_hbm,
          acc_sh, *bufs):
        buf_a, buf_b = bufs[:11], bufs[11:]
        cid = lax.axis_index("c")
        sid = lax.axis_index("s")
        _acc_zero(z_hbm, acc_sh, sid)
        off = cid * N
        lane_iota = lax.iota(jnp.int32, 16)

        def issue(ci, bufset):
            (sidx, didx, didx2, qrows, kvrows, arows, orows,
             sem1, sem2, sem3, semsc) = bufset
            eb = ci * EC
            pltpu.sync_copy(s2_hbm.at[pl.ds(eb, EC)], sidx)
            pltpu.sync_copy(d2_hbm.at[pl.ds(eb, EC)], didx)
            _adjust(sidx, sidx, off)
            _adjust(didx2, didx, off)
            pltpu.async_copy(q_hbm.at[didx2], qrows, sem1)
            pltpu.async_copy(kv_hbm.at[sidx], kvrows, sem2)
            pltpu.async_copy(at_hbm.at[pl.ds(eb, EC)], arows, sem3)

        def process(bufset, drain):
            (sidx, didx, didx2, qrows, kvrows, arows, orows,
             sem1, sem2, sem3, semsc) = bufset

            @pl.when(drain)
            def _():
                pltpu.make_async_copy(orows, acc_sh.at[didx], semsc).wait()
            pltpu.make_async_copy(q_hbm.at[didx2], qrows, sem1).wait()
            pltpu.make_async_copy(kv_hbm.at[sidx], kvrows, sem2).wait()
            pltpu.make_async_copy(at_hbm.at[pl.ds(0, EC)], arows, sem3).wait()

            @plsc.parallel_loop(0, EC, unroll=16)
            def ebody(e):
                attr_v = arows[e, :]
                atn = jnp.zeros((16,), jnp.float32)
                for h in range(HW):
                    pr = qrows[e, pl.ds(h * HD, HD)] \
                        * kvrows[e, pl.ds(h * HD, HD)] * attr_v
                    sh = _splat16(plsc.cumsum(pr), 15)
                    atn = jnp.where(lane_iota == h, sh, atn)
                w = jnp.exp(atn * 0.25)
                orows[e, pl.ds(VW, 16)] = w
                for h in range(HW):
                    wh = _splat16(w, h)
                    orows[e, pl.ds(h * HD, HD)] = \
                        wh * kvrows[e, pl.ds(VW + h * HD, HD)]

            pltpu.async_copy(orows, acc_sh.at[didx], semsc, add=True)

        issue(sid, buf_a)

        def guarded(k_, _):
            ci = sid + k_ * NS
            nxt = ci + NS

            @pl.when(ci < NCHUNK)
            def _():
                @pl.when(k_ % 2 == 0)
                def _():
                    @pl.when(nxt < NCHUNK)
                    def _():
                        issue(nxt, buf_b)
                    process(buf_a, k_ >= 2)

                @pl.when(k_ % 2 == 1)
                def _():
                    @pl.when(nxt < NCHUNK)
                    def _():
                        issue(nxt, buf_a)
                    process(buf_b, k_ >= 2)
            return 0

        lax.fori_loop(0, KMAX, guarded, 0)
        for bset in (buf_a, buf_b):
            pltpu.make_async_copy(bset[6], acc_sh.at[bset[1]], bset[10]).wait()
        _acc_writeback(acc_sh, out_hbm, cid, sid)

    return k(s2, d2, q2, kv2, attr, zeros).reshape(NC, NP, AW)


FP = 102400
FC = 128
FW = FP // NW
FNCH = FW // FC


def _fgather_sc(root3, idx_pad):
    mesh = plsc.VectorSubcoreMesh(**_MESH)

    @functools.partial(
        pl.kernel,
        out_type=jax.ShapeDtypeStruct((FP, ENC), jnp.float32),
        mesh=mesh,
        compiler_params=_SC_PARAMS,
        scratch_types=2 * [
            pltpu.VMEM((FC,), jnp.int32),
            pltpu.VMEM((FC, ENC), jnp.float32),
            pltpu.SemaphoreType.DMA,
            pltpu.SemaphoreType.DMA,
        ],
    )
    def k(t_hbm, i_hbm, out_hbm, *bufs):
        buf_a, buf_b = bufs[:4], bufs[4:]
        cid = lax.axis_index("c")
        sid = lax.axis_index("s")
        wid = sid * NC + cid
        fbase = wid * FW

        def issue(b, bufset, drain):
            iv, rv, sg, so = bufset

            @pl.when(drain)
            def _():
                pltpu.make_async_copy(rv, out_hbm.at[pl.ds(0, FC)], so).wait()
            pltpu.sync_copy(i_hbm.at[pl.ds(b, FC)], iv)
            pltpu.async_copy(t_hbm.at[iv], rv, sg)

        def process(b, bufset):
            iv, rv, sg, so = bufset
            pltpu.make_async_copy(t_hbm.at[iv], rv, sg).wait()
            pltpu.async_copy(rv, out_hbm.at[pl.ds(b, FC)], so)

        issue(fbase, buf_a, jnp.bool_(False))

        def chunk(ci, _):
            b = fbase + ci * FC

            @pl.when(ci % 2 == 0)
            def _():
                @pl.when(ci + 1 < FNCH)
                def _():
                    issue(b + FC, buf_b, ci >= 1)
                process(b, buf_a)

            @pl.when(ci % 2 == 1)
            def _():
                @pl.when(ci + 1 < FNCH)
                def _():
                    issue(b + FC, buf_a, ci >= 1)
                process(b, buf_b)
            return 0

        lax.fori_loop(0, FNCH, chunk, 0)
        pltpu.make_async_copy(buf_a[1], out_hbm.at[pl.ds(0, FC)],
                              buf_a[3]).wait()
        pltpu.make_async_copy(buf_b[1], out_hbm.at[pl.ds(0, FC)],
                              buf_b[3]).wait()

    return k(root3, idx_pad)


def kernel(root_features, feedback_features, feedback_index, fringe_features,
           root_to_fringe_index, root_edge_index, root_edge_attr, W_attn,
           W_ctx2x, W_x2g, W_qkv, W_gate, W_up, W_down, W_fgate, b_fgate,
           W_r2f, b_r2f, rms_node_w, rms_root_w, rms_ffn_w):
    p = jnp.kron(jnp.eye(CH, dtype=jnp.float32), jnp.ones((1, HD), jnp.float32))
    wqkv4 = W_qkv.reshape(ENC, SH, HD, 3)
    wq = wqkv4[..., 0].reshape(ENC, ENC)
    wkv = jnp.concatenate(
        [wqkv4[..., 1].reshape(ENC, ENC), wqkv4[..., 2].reshape(ENC, ENC)],
        axis=1)
    hff = W_gate.shape[1]
    wg = jnp.pad(W_gate, ((0, 0), (0, HFFP - hff)))
    wu = jnp.pad(W_up, ((0, 0), (0, HFFP - hff)))
    wd = jnp.pad(W_down, ((0, HFFP - hff), (0, 0)))

    src = feedback_index[0].astype(jnp.int32)
    dst = feedback_index[1].astype(jnp.int32)
    s2 = root_edge_index[0].astype(jnp.int32)
    d2 = root_edge_index[1].astype(jnp.int32)

    zeros = jnp.zeros((NP, AW), jnp.float32)
    idx_pad = jnp.pad(root_to_fringe_index.astype(jnp.int32), (0, FP - F))

    src2, xp2 = _pre1(feedback_features, root_features, W_attn, W_ctx2x)
    acc1 = _edge1_sc(src, dst, src2.reshape(NC * N, AW),
                     xp2.reshape(NC * N, 16), zeros)
    root1, q2, kv2 = _mid(acc1, root_features, W_x2g, rms_node_w, wq, wkv, p)
    acc2 = _edge2_sc(s2, d2, q2.reshape(NC * N, VW),
                     kv2.reshape(NC * N, 2 * VW), root_edge_attr, zeros)
    root3 = _post(acc2, root1, rms_root_w, wg, wu, wd, rms_ffn_w, p)
    gathered = _fgather_sc(root3, idx_pad)
    fringe_out = _fringe(fringe_features, gathered, W_fgate, b_fgate, W_r2f,
                         b_r2f)
    return (root3, fringe_out)


# speedup vs baseline: 1.2535x; 1.2535x over previous
import functools
from math import sqrt

import jax
import jax.numpy as jnp
from jax import lax
from jax.experimental import pallas as pl
from jax.experimental.pallas import tpu as pltpu
from jax.experimental.pallas import tpu_sc as plsc

N = 10000
E = 320000
F = 100000
ENC = 128
DEC = 128
CH = 8
SH = 8
HD = 16
NEG_SLOPE = 0.1
HFFP = 384

ROW_BLK = 1000

NC, NS = 2, 16
NP = 10240
NW = NC * NS
EC = 128
NCHUNK = E // EC
KMAX = (NCHUNK + NS - 1) // NS
TR = NP // NS
HW = 4
VW = HW * HD
AW = VW + 16


def _swish(x):
    return x * jax.nn.sigmoid(x)


def _rms_norm(x, w):
    return x / jnp.sqrt(jnp.mean(x * x, axis=-1, keepdims=True) + 1e-6) * w


def _pre1_body(fb_ref, root_ref, wa_ref, wc_ref, src_t_ref, xp_t_ref):
    fb = fb_ref[...]
    root = root_ref[...]
    ctxp = jnp.dot(fb, wc_ref[...], preferred_element_type=jnp.float32)
    cp = jnp.dot(fb, wa_ref[...][:DEC], preferred_element_type=jnp.float32)
    xp = jnp.dot(root, wa_ref[...][DEC:], preferred_element_type=jnp.float32)
    z12 = jnp.zeros((fb.shape[0], 12), jnp.float32)
    for c in range(NC):
        src_t_ref[c] = jnp.concatenate(
            [ctxp[:, c * VW:(c + 1) * VW], cp[:, c * HW:(c + 1) * HW], z12],
            axis=1)
        xp_t_ref[c] = jnp.concatenate(
            [xp[:, c * HW:(c + 1) * HW], z12], axis=1)


def _pre1(fb, root, w_attn, w_ctx2x):
    grid = (N // ROW_BLK,)
    return pl.pallas_call(
        _pre1_body,
        grid=grid,
        in_specs=[
            pl.BlockSpec((ROW_BLK, DEC), lambda i: (i, 0)),
            pl.BlockSpec((ROW_BLK, ENC), lambda i: (i, 0)),
            pl.BlockSpec((DEC + ENC, CH), lambda i: (0, 0)),
            pl.BlockSpec((DEC, ENC), lambda i: (0, 0)),
        ],
        out_specs=[
            pl.BlockSpec((NC, ROW_BLK, AW), lambda i: (0, i, 0)),
            pl.BlockSpec((NC, ROW_BLK, 16), lambda i: (0, i, 0)),
        ],
        out_shape=[
            jax.ShapeDtypeStruct((NC, N, AW), jnp.float32),
            jax.ShapeDtypeStruct((NC, N, 16), jnp.float32),
        ],
    )(fb, root, w_attn, w_ctx2x)


def _combine(acc_ref):
    unm = jnp.concatenate([acc_ref[0][:, :VW], acc_ref[1][:, :VW]], axis=1)
    s = jnp.concatenate(
        [acc_ref[0][:, VW:VW + HW], acc_ref[1][:, VW:VW + HW]], axis=1)
    has_in = acc_ref[0][:, VW + HW:VW + HW + 1] > 0
    return unm, s, has_in


def _mid_body(acc_ref, root_ref, wx2g_ref, wrms_ref, wq_ref, wkv_ref, p_ref,
              root1_ref, q2_ref, kv2_ref):
    root = root_ref[...]
    p = p_ref[...]
    unm, s, has_in = _combine(acc_ref)
    inv = 1.0 / (s + 1e-16)
    mess = unm * jnp.dot(inv, p, preferred_element_type=jnp.float32)
    gates = jnp.dot(
        jnp.dot(root, wx2g_ref[...], preferred_element_type=jnp.float32),
        p, preferred_element_type=jnp.float32)
    gates = jnp.where(jnp.broadcast_to(has_in, gates.shape), gates, 1.0)
    out = gates * root + (1.0 - gates) * mess
    root1 = _rms_norm(root + out, wrms_ref[...][0])
    root1_ref[...] = root1
    q = jnp.dot(root1, wq_ref[...], preferred_element_type=jnp.float32)
    kv = jnp.dot(root1, wkv_ref[...], preferred_element_type=jnp.float32)
    for c in range(NC):
        q2_ref[c] = q[:, c * VW:(c + 1) * VW]
        kv2_ref[c] = jnp.concatenate(
            [kv[:, c * VW:(c + 1) * VW],
             kv[:, ENC + c * VW:ENC + (c + 1) * VW]], axis=1)


def _mid(acc, root, w_x2g, rms_node_w, wq, wkv, p):
    grid = (N // ROW_BLK,)
    return pl.pallas_call(
        _mid_body,
        grid=grid,
        in_specs=[
            pl.BlockSpec((NC, ROW_BLK, AW), lambda i: (0, i, 0)),
            pl.BlockSpec((ROW_BLK, ENC), lambda i: (i, 0)),
            pl.BlockSpec((ENC, CH), lambda i: (0, 0)),
            pl.BlockSpec((1, ENC), lambda i: (0, 0)),
            pl.BlockSpec((ENC, ENC), lambda i: (0, 0)),
            pl.BlockSpec((ENC, 2 * ENC), lambda i: (0, 0)),
            pl.BlockSpec((CH, ENC), lambda i: (0, 0)),
        ],
        out_specs=[
            pl.BlockSpec((ROW_BLK, ENC), lambda i: (i, 0)),
            pl.BlockSpec((NC, ROW_BLK, VW), lambda i: (0, i, 0)),
            pl.BlockSpec((NC, ROW_BLK, 2 * VW), lambda i: (0, i, 0)),
        ],
        out_shape=[
            jax.ShapeDtypeStruct((N, ENC), jnp.float32),
            jax.ShapeDtypeStruct((NC, N, VW), jnp.float32),
            jax.ShapeDtypeStruct((NC, N, 2 * VW), jnp.float32),
        ],
    )(acc, root, w_x2g, rms_node_w.reshape(1, ENC), wq, wkv, p)


def _post_body(acc_ref, root1_ref, wr_ref, wg_ref, wu_ref, wd_ref, wf_ref,
               p_ref, root3_ref):
    root1 = root1_ref[...]
    unm, s, _ = _combine(acc_ref)
    inv = 1.0 / (s + 1e-16)
    mha = unm * jnp.dot(inv, p_ref[...], preferred_element_type=jnp.float32)
    root2 = _rms_norm(root1 + mha, wr_ref[...][0])
    gate = jnp.dot(root2, wg_ref[...], preferred_element_type=jnp.float32)
    up = jnp.dot(root2, wu_ref[...], preferred_element_type=jnp.float32)
    ffn = jnp.dot(_swish(gate) * up, wd_ref[...], preferred_element_type=jnp.float32)
    root3_ref[...] = _rms_norm(root2 + ffn, wf_ref[...][0])


def _post(acc, root1, rms_root_w, wg, wu, wd, rms_ffn_w, p):
    grid = (N // ROW_BLK,)
    return pl.pallas_call(
        _post_body,
        grid=grid,
        in_specs=[
            pl.BlockSpec((NC, ROW_BLK, AW), lambda i: (0, i, 0)),
            pl.BlockSpec((ROW_BLK, ENC), lambda i: (i, 0)),
            pl.BlockSpec((1, ENC), lambda i: (0, 0)),
            pl.BlockSpec((ENC, HFFP), lambda i: (0, 0)),
            pl.BlockSpec((ENC, HFFP), lambda i: (0, 0)),
            pl.BlockSpec((HFFP, ENC), lambda i: (0, 0)),
            pl.BlockSpec((1, ENC), lambda i: (0, 0)),
            pl.BlockSpec((CH, ENC), lambda i: (0, 0)),
        ],
        out_specs=pl.BlockSpec((ROW_BLK, ENC), lambda i: (i, 0)),
        out_shape=jax.ShapeDtypeStruct((N, ENC), jnp.float32),
    )(acc, root1, rms_root_w.reshape(1, ENC), wg, wu, wd,
      rms_ffn_w.reshape(1, ENC), p)


FR_BLK = 1000


def _fringe_body(fr_ref, gr_ref, wfg_ref, bfg_ref, wrf_ref, brf_ref, out_ref):
    fr = fr_ref[...]
    fg = _swish(jnp.dot(fr, wfg_ref[...], preferred_element_type=jnp.float32)
                + bfg_ref[...][0])
    r2f = jnp.dot(gr_ref[...], wrf_ref[...], preferred_element_type=jnp.float32) \
        + brf_ref[...][0]
    out_ref[...] = r2f * fg


def _fringe(fr, gathered, w_fgate, b_fgate, w_r2f, b_r2f):
    grid = (F // FR_BLK,)
    return pl.pallas_call(
        _fringe_body,
        grid=grid,
        in_specs=[
            pl.BlockSpec((FR_BLK, DEC), lambda i: (i, 0)),
            pl.BlockSpec((FR_BLK, ENC), lambda i: (i, 0)),
            pl.BlockSpec((DEC, DEC), lambda i: (0, 0)),
            pl.BlockSpec((1, DEC), lambda i: (0, 0)),
            pl.BlockSpec((ENC, DEC), lambda i: (0, 0)),
            pl.BlockSpec((1, DEC), lambda i: (0, 0)),
        ],
        out_specs=pl.BlockSpec((FR_BLK, DEC), lambda i: (i, 0)),
        out_shape=jax.ShapeDtypeStruct((F, DEC), jnp.float32),
    )(fr, gathered, w_fgate, b_fgate.reshape(1, DEC), w_r2f,
      b_r2f.reshape(1, DEC))


_SC_PARAMS = pltpu.CompilerParams(use_tc_tiling_on_sc=False,
                                  needs_layout_passes=False)
_MESH = dict(core_axis_name="c", subcore_axis_name="s", num_cores=NC,
             num_subcores=NS)


def _splat16(v, i):
    idx = jnp.full((16, 1), i, jnp.int32)
    return lax.gather(
        v, idx,
        lax.GatherDimensionNumbers(offset_dims=(), collapsed_slice_dims=(0,),
                                   start_index_map=(0,)),
        (1,), mode=lax.GatherScatterMode.PROMISE_IN_BOUNDS)


def _adjust(dst_idx_ref, src_idx_ref, off):
    for j in range(EC // 16):
        sl = pl.ds(j * 16, 16)
        dst_idx_ref[sl] = src_idx_ref[sl] + off


def _acc_writeback(acc_sh, out_hbm, cid, sid):
    plsc.subcore_barrier()
    base = sid * TR
    pltpu.sync_copy(acc_sh.at[pl.ds(base, TR)],
                    out_hbm.at[pl.ds(cid * NP + base, TR)])


def _acc_zero(zeros_hbm, acc_sh, sid):
    base = sid * TR
    pltpu.sync_copy(zeros_hbm.at[pl.ds(base, TR)], acc_sh.at[pl.ds(base, TR)])
    plsc.subcore_barrier()


def _edge1_sc(fi, src2, xp2, zeros):
    mesh = plsc.VectorSubcoreMesh(**_MESH)

    @functools.partial(
        pl.kernel,
        out_type=jax.ShapeDtypeStruct((NC * NP, AW), jnp.float32),
        mesh=mesh,
        compiler_params=_SC_PARAMS,
        scratch_types=[pltpu.VMEM_SHARED((NP, AW), jnp.float32)] + 2 * [
            pltpu.VMEM((2, EC), jnp.int32),
            pltpu.VMEM((EC,), jnp.int32),
            pltpu.VMEM((EC,), jnp.int32),
            pltpu.VMEM((EC, AW), jnp.float32),
            pltpu.VMEM((EC, 16), jnp.float32),
            pltpu.SemaphoreType.DMA,
            pltpu.SemaphoreType.DMA,
            pltpu.SemaphoreType.DMA,
        ],
    )
    def k(fi_hbm, st_hbm, xp_hbm, z_hbm, out_hbm, acc_sh, *bufs):
        buf_a, buf_b = bufs[:8], bufs[8:]
        cid = lax.axis_index("c")
        sid = lax.axis_index("s")
        _acc_zero(z_hbm, acc_sh, sid)
        off = cid * N

        def issue(ci, bufset, drain):
            sd, sidx, didx2, rows, xpr, sem1, sem2, semsc = bufset
            if drain is not None:
                @pl.when(drain)
                def _():
                    pltpu.make_async_copy(rows, acc_sh.at[sd.at[1]],
                                          semsc).wait()
            eb = ci * EC
            pltpu.sync_copy(fi_hbm.at[:, pl.ds(eb, EC)], sd)
            _adjust(sidx, sd.at[0], off)
            _adjust(didx2, sd.at[1], off)
            pltpu.async_copy(st_hbm.at[sidx], rows, sem1)
            pltpu.async_copy(xp_hbm.at[didx2], xpr, sem2)

        def process(bufset):
            sd, sidx, didx2, rows, xpr, sem1, sem2, semsc = bufset
            didx = sd.at[1]
            pltpu.make_async_copy(st_hbm.at[sidx], rows, sem1).wait()
            pltpu.make_async_copy(xp_hbm.at[didx2], xpr, sem2).wait()

            @plsc.parallel_loop(0, EC, unroll=8)
            def ebody(e):
                cp = rows[e, pl.ds(VW, 16)]
                z = cp + xpr[e, :]
                a = jnp.maximum(z, NEG_SLOPE * z)
                w = jnp.exp(a)
                rows[e, pl.ds(VW, 16)] = w
                for h in range(HW):
                    wh = _splat16(w, h)
                    rows[e, pl.ds(h * HD, HD)] = wh * rows[e, pl.ds(h * HD, HD)]

            pltpu.async_copy(rows, acc_sh.at[didx], semsc, add=True)

        issue(sid, buf_a, None)

        def guarded(k_, _):
            ci = sid + k_ * NS
            nxt = ci + NS

            @pl.when(ci < NCHUNK)
            def _():
                @pl.when(k_ % 2 == 0)
                def _():
                    @pl.when(nxt < NCHUNK)
                    def _():
                        issue(nxt, buf_b, k_ >= 1)
                    process(buf_a)

                @pl.when(k_ % 2 == 1)
                def _():
                    @pl.when(nxt < NCHUNK)
                    def _():
                        issue(nxt, buf_a, k_ >= 1)
                    process(buf_b)
            return 0

        lax.fori_loop(0, KMAX, guarded, 0)
        for bset in (buf_a, buf_b):
            pltpu.make_async_copy(bset[3], acc_sh.at[bset[0].at[1]],
                                  bset[7]).wait()
        _acc_writeback(acc_sh, out_hbm, cid, sid)

    return k(fi, src2, xp2, zeros).reshape(NC, NP, AW)


def _edge2_sc(fi, q2, kv2, attr, zeros):
    mesh = plsc.VectorSubcoreMesh(**_MESH)

    @functools.partial(
        pl.kernel,
        out_type=jax.ShapeDtypeStruct((NC * NP, AW), jnp.float32),
        mesh=mesh,
        compiler_params=_SC_PARAMS,
        scratch_types=[
            pltpu.VMEM_SHARED((NP, AW), jnp.float32),
        ] + 2 * [
            pltpu.VMEM((2, EC), jnp.int32),
            pltpu.VMEM((EC,), jnp.int32),
            pltpu.VMEM((EC,), jnp.int32),
            pltpu.VMEM((EC, VW), jnp.float32),
            pltpu.VMEM((EC, 2 * VW), jnp.float32),
            pltpu.VMEM((EC, 16), jnp.float32),
            pltpu.VMEM((EC, AW), jnp.float32),
            pltpu.SemaphoreType.DMA,
            pltpu.SemaphoreType.DMA,
            pltpu.SemaphoreType.DMA,
            pltpu.SemaphoreType.DMA,
        ],
    )
    def k(fi_hbm, q_hbm, kv_hbm, at_hbm, z_hbm, out_hbm,
          acc_sh, *bufs):
        buf_a, buf_b = bufs[:11], bufs[11:]
        cid = lax.axis_index("c")
        sid = lax.axis_index("s")
        _acc_zero(z_hbm, acc_sh, sid)
        off = cid * N
        lane_iota = lax.iota(jnp.int32, 16)

        def issue(ci, bufset):
            (sd, sidx, didx2, qrows, kvrows, arows, orows,
             sem1, sem2, sem3, semsc) = bufset
            eb = ci * EC
            pltpu.sync_copy(fi_hbm.at[:, pl.ds(eb, EC)], sd)
            _adjust(sidx, sd.at[0], off)
            _adjust(didx2, sd.at[1], off)
            pltpu.async_copy(q_hbm.at[didx2], qrows, sem1)
            pltpu.async_copy(kv_hbm.at[sidx], kvrows, sem2)
            pltpu.async_copy(at_hbm.at[pl.ds(eb, EC)], arows, sem3)

        def process(bufset, drain):
            (sd, sidx, didx2, qrows, kvrows, arows, orows,
             sem1, sem2, sem3, semsc) = bufset
            didx = sd.at[1]

            @pl.when(drain)
            def _():
                pltpu.make_async_copy(orows, acc_sh.at[didx], semsc).wait()
            pltpu.make_async_copy(q_hbm.at[didx2], qrows, sem1).wait()
            pltpu.make_async_copy(kv_hbm.at[sidx], kvrows, sem2).wait()
            pltpu.make_async_copy(at_hbm.at[pl.ds(0, EC)], arows, sem3).wait()

            @plsc.parallel_loop(0, EC, unroll=8)
            def ebody(e):
                attr_v = arows[e, :]
                atn = jnp.zeros((16,), jnp.float32)
                for h in range(HW):
                    pr = qrows[e, pl.ds(h * HD, HD)] \
                        * kvrows[e, pl.ds(h * HD, HD)] * attr_v
                    sh = _splat16(plsc.cumsum(pr), 15)
                    atn = jnp.where(lane_iota == h, sh, atn)
                w = jnp.exp(atn * 0.25)
                orows[e, pl.ds(VW, 16)] = w
                for h in range(HW):
                    wh = _splat16(w, h)
                    orows[e, pl.ds(h * HD, HD)] = \
                        wh * kvrows[e, pl.ds(VW + h * HD, HD)]

            pltpu.async_copy(orows, acc_sh.at[didx], semsc, add=True)

        issue(sid, buf_a)

        def guarded(k_, _):
            ci = sid + k_ * NS
            nxt = ci + NS

            @pl.when(ci < NCHUNK)
            def _():
                @pl.when(k_ % 2 == 0)
                def _():
                    @pl.when(nxt < NCHUNK)
                    def _():
                        issue(nxt, buf_b)
                    process(buf_a, k_ >= 2)

                @pl.when(k_ % 2 == 1)
                def _():
                    @pl.when(nxt < NCHUNK)
                    def _():
                        issue(nxt, buf_a)
                    process(buf_b, k_ >= 2)
            return 0

        lax.fori_loop(0, KMAX, guarded, 0)
        for bset in (buf_a, buf_b):
            pltpu.make_async_copy(bset[6], acc_sh.at[bset[0].at[1]],
                                  bset[10]).wait()
        _acc_writeback(acc_sh, out_hbm, cid, sid)

    return k(fi, q2, kv2, attr, zeros).reshape(NC, NP, AW)


FP = 102400
FC = 128
FW = FP // NW
FNCH = FW // FC


def _fgather_sc(root3, idx_pad):
    mesh = plsc.VectorSubcoreMesh(**_MESH)

    @functools.partial(
        pl.kernel,
        out_type=jax.ShapeDtypeStruct((FP, ENC), jnp.float32),
        mesh=mesh,
        compiler_params=_SC_PARAMS,
        scratch_types=2 * [
            pltpu.VMEM((FC,), jnp.int32),
            pltpu.VMEM((FC, ENC), jnp.float32),
            pltpu.SemaphoreType.DMA,
            pltpu.SemaphoreType.DMA,
        ],
    )
    def k(t_hbm, i_hbm, out_hbm, *bufs):
        buf_a, buf_b = bufs[:4], bufs[4:]
        cid = lax.axis_index("c")
        sid = lax.axis_index("s")
        wid = sid * NC + cid
        fbase = wid * FW

        def issue(b, bufset, drain):
            iv, rv, sg, so = bufset

            @pl.when(drain)
            def _():
                pltpu.make_async_copy(rv, out_hbm.at[pl.ds(0, FC)], so).wait()
            pltpu.sync_copy(i_hbm.at[pl.ds(b, FC)], iv)
            pltpu.async_copy(t_hbm.at[iv], rv, sg)

        def process(b, bufset):
            iv, rv, sg, so = bufset
            pltpu.make_async_copy(t_hbm.at[iv], rv, sg).wait()
            pltpu.async_copy(rv, out_hbm.at[pl.ds(b, FC)], so)

        issue(fbase, buf_a, jnp.bool_(False))

        def chunk(ci, _):
            b = fbase + ci * FC

            @pl.when(ci % 2 == 0)
            def _():
                @pl.when(ci + 1 < FNCH)
                def _():
                    issue(b + FC, buf_b, ci >= 1)
                process(b, buf_a)

            @pl.when(ci % 2 == 1)
            def _():
                @pl.when(ci + 1 < FNCH)
                def _():
                    issue(b + FC, buf_a, ci >= 1)
                process(b, buf_b)
            return 0

        lax.fori_loop(0, FNCH, chunk, 0)
        pltpu.make_async_copy(buf_a[1], out_hbm.at[pl.ds(0, FC)],
                              buf_a[3]).wait()
        pltpu.make_async_copy(buf_b[1], out_hbm.at[pl.ds(0, FC)],
                              buf_b[3]).wait()

    return k(root3, idx_pad)


def kernel(root_features, feedback_features, feedback_index, fringe_features,
           root_to_fringe_index, root_edge_index, root_edge_attr, W_attn,
           W_ctx2x, W_x2g, W_qkv, W_gate, W_up, W_down, W_fgate, b_fgate,
           W_r2f, b_r2f, rms_node_w, rms_root_w, rms_ffn_w):
    p = jnp.kron(jnp.eye(CH, dtype=jnp.float32), jnp.ones((1, HD), jnp.float32))
    wqkv4 = W_qkv.reshape(ENC, SH, HD, 3)
    wq = wqkv4[..., 0].reshape(ENC, ENC)
    wkv = jnp.concatenate(
        [wqkv4[..., 1].reshape(ENC, ENC), wqkv4[..., 2].reshape(ENC, ENC)],
        axis=1)
    hff = W_gate.shape[1]
    wg = jnp.pad(W_gate, ((0, 0), (0, HFFP - hff)))
    wu = jnp.pad(W_up, ((0, 0), (0, HFFP - hff)))
    wd = jnp.pad(W_down, ((0, HFFP - hff), (0, 0)))

    fi1 = feedback_index.astype(jnp.int32)
    fi2 = root_edge_index.astype(jnp.int32)

    zeros = jnp.zeros((NP, AW), jnp.float32)
    idx_pad = jnp.pad(root_to_fringe_index.astype(jnp.int32), (0, FP - F))

    src2, xp2 = _pre1(feedback_features, root_features, W_attn, W_ctx2x)
    acc1 = _edge1_sc(fi1, src2.reshape(NC * N, AW),
                     xp2.reshape(NC * N, 16), zeros)
    root1, q2, kv2 = _mid(acc1, root_features, W_x2g, rms_node_w, wq, wkv, p)
    acc2 = _edge2_sc(fi2, q2.reshape(NC * N, VW),
                     kv2.reshape(NC * N, 2 * VW), root_edge_attr, zeros)
    root3 = _post(acc2, root1, rms_root_w, wg, wu, wd, rms_ffn_w, p)
    gathered = _fgather_sc(root3, idx_pad)
    fringe_out = _fringe(fringe_features, gathered, W_fgate, b_fgate, W_r2f,
                         b_r2f)
    return (root3, fringe_out)


# speedup vs baseline: 1.2650x; 1.0091x over previous
import functools
from math import sqrt

import jax
import jax.numpy as jnp
from jax import lax
from jax.experimental import pallas as pl
from jax.experimental.pallas import tpu as pltpu
from jax.experimental.pallas import tpu_sc as plsc

N = 10000
E = 320000
F = 100000
ENC = 128
DEC = 128
CH = 8
SH = 8
HD = 16
NEG_SLOPE = 0.1
HFFP = 384

ROW_BLK = 1000

NC, NS = 2, 16
NP = 10240
NW = NC * NS
EC = 128
NCHUNK = E // EC
KMAX = (NCHUNK + NS - 1) // NS
TR = NP // NS
HW = 4
VW = HW * HD
AW = VW + 16


def _swish(x):
    return x * jax.nn.sigmoid(x)


def _rms_norm(x, w):
    return x / jnp.sqrt(jnp.mean(x * x, axis=-1, keepdims=True) + 1e-6) * w


def _pre1_body(fb_ref, root_ref, wa_ref, wc_ref, src_t_ref, xp_t_ref):
    fb = fb_ref[...]
    root = root_ref[...]
    ctxp = jnp.dot(fb, wc_ref[...], preferred_element_type=jnp.float32)
    cp = jnp.dot(fb, wa_ref[...][:DEC], preferred_element_type=jnp.float32)
    xp = jnp.dot(root, wa_ref[...][DEC:], preferred_element_type=jnp.float32)
    z12 = jnp.zeros((fb.shape[0], 12), jnp.float32)
    for c in range(NC):
        src_t_ref[c] = jnp.concatenate(
            [ctxp[:, c * VW:(c + 1) * VW], cp[:, c * HW:(c + 1) * HW], z12],
            axis=1)
        xp_t_ref[c] = jnp.concatenate(
            [xp[:, c * HW:(c + 1) * HW], z12], axis=1)


def _pre1(fb, root, w_attn, w_ctx2x):
    grid = (N // ROW_BLK,)
    return pl.pallas_call(
        _pre1_body,
        grid=grid,
        in_specs=[
            pl.BlockSpec((ROW_BLK, DEC), lambda i: (i, 0)),
            pl.BlockSpec((ROW_BLK, ENC), lambda i: (i, 0)),
            pl.BlockSpec((DEC + ENC, CH), lambda i: (0, 0)),
            pl.BlockSpec((DEC, ENC), lambda i: (0, 0)),
        ],
        out_specs=[
            pl.BlockSpec((NC, ROW_BLK, AW), lambda i: (0, i, 0)),
            pl.BlockSpec((NC, ROW_BLK, 16), lambda i: (0, i, 0)),
        ],
        out_shape=[
            jax.ShapeDtypeStruct((NC, N, AW), jnp.float32),
            jax.ShapeDtypeStruct((NC, N, 16), jnp.float32),
        ],
    )(fb, root, w_attn, w_ctx2x)


def _combine(acc_ref):
    unm = jnp.concatenate([acc_ref[0][:, :VW], acc_ref[1][:, :VW]], axis=1)
    s = jnp.concatenate(
        [acc_ref[0][:, VW:VW + HW], acc_ref[1][:, VW:VW + HW]], axis=1)
    has_in = acc_ref[0][:, VW + HW:VW + HW + 1] > 0
    return unm, s, has_in


def _mid_body(acc_ref, root_ref, wx2g_ref, wrms_ref, wq_ref, wkv_ref, p_ref,
              root1_ref, q2_ref, kv2_ref):
    root = root_ref[...]
    p = p_ref[...]
    unm, s, has_in = _combine(acc_ref)
    inv = 1.0 / (s + 1e-16)
    mess = unm * jnp.dot(inv, p, preferred_element_type=jnp.float32)
    gates = jnp.dot(
        jnp.dot(root, wx2g_ref[...], preferred_element_type=jnp.float32),
        p, preferred_element_type=jnp.float32)
    gates = jnp.where(jnp.broadcast_to(has_in, gates.shape), gates, 1.0)
    out = gates * root + (1.0 - gates) * mess
    root1 = _rms_norm(root + out, wrms_ref[...][0])
    root1_ref[...] = root1
    q = jnp.dot(root1, wq_ref[...], preferred_element_type=jnp.float32)
    kv = jnp.dot(root1, wkv_ref[...], preferred_element_type=jnp.float32)
    for c in range(NC):
        q2_ref[c] = q[:, c * VW:(c + 1) * VW]
        kv2_ref[c] = jnp.concatenate(
            [kv[:, c * VW:(c + 1) * VW],
             kv[:, ENC + c * VW:ENC + (c + 1) * VW]], axis=1)


def _mid(acc, root, w_x2g, rms_node_w, wq, wkv, p):
    grid = (N // ROW_BLK,)
    return pl.pallas_call(
        _mid_body,
        grid=grid,
        in_specs=[
            pl.BlockSpec((NC, ROW_BLK, AW), lambda i: (0, i, 0)),
            pl.BlockSpec((ROW_BLK, ENC), lambda i: (i, 0)),
            pl.BlockSpec((ENC, CH), lambda i: (0, 0)),
            pl.BlockSpec((1, ENC), lambda i: (0, 0)),
            pl.BlockSpec((ENC, ENC), lambda i: (0, 0)),
            pl.BlockSpec((ENC, 2 * ENC), lambda i: (0, 0)),
            pl.BlockSpec((CH, ENC), lambda i: (0, 0)),
        ],
        out_specs=[
            pl.BlockSpec((ROW_BLK, ENC), lambda i: (i, 0)),
            pl.BlockSpec((NC, ROW_BLK, VW), lambda i: (0, i, 0)),
            pl.BlockSpec((NC, ROW_BLK, 2 * VW), lambda i: (0, i, 0)),
        ],
        out_shape=[
            jax.ShapeDtypeStruct((N, ENC), jnp.float32),
            jax.ShapeDtypeStruct((NC, N, VW), jnp.float32),
            jax.ShapeDtypeStruct((NC, N, 2 * VW), jnp.float32),
        ],
    )(acc, root, w_x2g, rms_node_w.reshape(1, ENC), wq, wkv, p)


def _post_body(acc_ref, root1_ref, wr_ref, wg_ref, wu_ref, wd_ref, wf_ref,
               p_ref, wrf_ref, brf_ref, root3_ref, r2f_ref):
    root1 = root1_ref[...]
    unm, s, _ = _combine(acc_ref)
    inv = 1.0 / (s + 1e-16)
    mha = unm * jnp.dot(inv, p_ref[...], preferred_element_type=jnp.float32)
    root2 = _rms_norm(root1 + mha, wr_ref[...][0])
    gate = jnp.dot(root2, wg_ref[...], preferred_element_type=jnp.float32)
    up = jnp.dot(root2, wu_ref[...], preferred_element_type=jnp.float32)
    ffn = jnp.dot(_swish(gate) * up, wd_ref[...], preferred_element_type=jnp.float32)
    root3 = _rms_norm(root2 + ffn, wf_ref[...][0])
    root3_ref[...] = root3
    r2f_ref[...] = jnp.dot(root3, wrf_ref[...],
                           preferred_element_type=jnp.float32) + brf_ref[...][0]


def _post(acc, root1, rms_root_w, wg, wu, wd, rms_ffn_w, p, wrf, brf):
    grid = (N // ROW_BLK,)
    return pl.pallas_call(
        _post_body,
        grid=grid,
        in_specs=[
            pl.BlockSpec((NC, ROW_BLK, AW), lambda i: (0, i, 0)),
            pl.BlockSpec((ROW_BLK, ENC), lambda i: (i, 0)),
            pl.BlockSpec((1, ENC), lambda i: (0, 0)),
            pl.BlockSpec((ENC, HFFP), lambda i: (0, 0)),
            pl.BlockSpec((ENC, HFFP), lambda i: (0, 0)),
            pl.BlockSpec((HFFP, ENC), lambda i: (0, 0)),
            pl.BlockSpec((1, ENC), lambda i: (0, 0)),
            pl.BlockSpec((CH, ENC), lambda i: (0, 0)),
            pl.BlockSpec((ENC, DEC), lambda i: (0, 0)),
            pl.BlockSpec((1, DEC), lambda i: (0, 0)),
        ],
        out_specs=[
            pl.BlockSpec((ROW_BLK, ENC), lambda i: (i, 0)),
            pl.BlockSpec((ROW_BLK, DEC), lambda i: (i, 0)),
        ],
        out_shape=[
            jax.ShapeDtypeStruct((N, ENC), jnp.float32),
            jax.ShapeDtypeStruct((N, DEC), jnp.float32),
        ],
    )(acc, root1, rms_root_w.reshape(1, ENC), wg, wu, wd,
      rms_ffn_w.reshape(1, ENC), p, wrf, brf.reshape(1, DEC))


FR_BLK = 1000


def _fgate_body(fr_ref, wfg_ref, bfg_ref, out_ref):
    out_ref[...] = _swish(
        jnp.dot(fr_ref[...], wfg_ref[...], preferred_element_type=jnp.float32)
        + bfg_ref[...][0])


def _fgate(fr, w_fgate, b_fgate):
    grid = (F // FR_BLK,)
    return pl.pallas_call(
        _fgate_body,
        grid=grid,
        in_specs=[
            pl.BlockSpec((FR_BLK, DEC), lambda i: (i, 0)),
            pl.BlockSpec((DEC, DEC), lambda i: (0, 0)),
            pl.BlockSpec((1, DEC), lambda i: (0, 0)),
        ],
        out_specs=pl.BlockSpec((FR_BLK, DEC), lambda i: (i, 0)),
        out_shape=jax.ShapeDtypeStruct((F, DEC), jnp.float32),
    )(fr, w_fgate, b_fgate.reshape(1, DEC))


def _fringe_body(gr_ref, fg_ref, out_ref):
    out_ref[...] = gr_ref[...] * fg_ref[...]


def _fringe(gathered_pad, fg):
    grid = (F // FR_BLK,)
    return pl.pallas_call(
        _fringe_body,
        grid=grid,
        in_specs=[
            pl.BlockSpec((FR_BLK, ENC), lambda i: (i, 0)),
            pl.BlockSpec((FR_BLK, DEC), lambda i: (i, 0)),
        ],
        out_specs=pl.BlockSpec((FR_BLK, DEC), lambda i: (i, 0)),
        out_shape=jax.ShapeDtypeStruct((F, DEC), jnp.float32),
    )(gathered_pad, fg)


_SC_PARAMS = pltpu.CompilerParams(use_tc_tiling_on_sc=False,
                                  needs_layout_passes=False)
_MESH = dict(core_axis_name="c", subcore_axis_name="s", num_cores=NC,
             num_subcores=NS)


def _splat16(v, i):
    idx = jnp.full((16, 1), i, jnp.int32)
    return lax.gather(
        v, idx,
        lax.GatherDimensionNumbers(offset_dims=(), collapsed_slice_dims=(0,),
                                   start_index_map=(0,)),
        (1,), mode=lax.GatherScatterMode.PROMISE_IN_BOUNDS)


def _adjust(dst_idx_ref, src_idx_ref, off):
    for j in range(EC // 16):
        sl = pl.ds(j * 16, 16)
        dst_idx_ref[sl] = src_idx_ref[sl] + off


def _acc_writeback(acc_sh, out_hbm, cid, sid):
    plsc.subcore_barrier()
    base = sid * TR
    pltpu.sync_copy(acc_sh.at[pl.ds(base, TR)],
                    out_hbm.at[pl.ds(cid * NP + base, TR)])


def _acc_zero(zeros_hbm, acc_sh, sid):
    base = sid * TR
    pltpu.sync_copy(zeros_hbm.at[pl.ds(base, TR)], acc_sh.at[pl.ds(base, TR)])
    plsc.subcore_barrier()


def _edge1_sc(fi, src2, xp2, zeros):
    mesh = plsc.VectorSubcoreMesh(**_MESH)

    @functools.partial(
        pl.kernel,
        out_type=jax.ShapeDtypeStruct((NC * NP, AW), jnp.float32),
        mesh=mesh,
        compiler_params=_SC_PARAMS,
        scratch_types=[pltpu.VMEM_SHARED((NP, AW), jnp.float32)] + 2 * [
            pltpu.VMEM((2, EC), jnp.int32),
            pltpu.VMEM((EC,), jnp.int32),
            pltpu.VMEM((EC,), jnp.int32),
            pltpu.VMEM((EC, AW), jnp.float32),
            pltpu.VMEM((EC, 16), jnp.float32),
            pltpu.SemaphoreType.DMA,
            pltpu.SemaphoreType.DMA,
            pltpu.SemaphoreType.DMA,
        ],
    )
    def k(fi_hbm, st_hbm, xp_hbm, z_hbm, out_hbm, acc_sh, *bufs):
        buf_a, buf_b = bufs[:8], bufs[8:]
        cid = lax.axis_index("c")
        sid = lax.axis_index("s")
        _acc_zero(z_hbm, acc_sh, sid)
        off = cid * N

        def issue(ci, bufset, drain):
            sd, sidx, didx2, rows, xpr, sem1, sem2, semsc = bufset
            if drain is not None:
                @pl.when(drain)
                def _():
                    pltpu.make_async_copy(rows, acc_sh.at[sd.at[1]],
                                          semsc).wait()
            eb = ci * EC
            pltpu.sync_copy(fi_hbm.at[:, pl.ds(eb, EC)], sd)
            _adjust(sidx, sd.at[0], off)
            _adjust(didx2, sd.at[1], off)
            pltpu.async_copy(st_hbm.at[sidx], rows, sem1)
            pltpu.async_copy(xp_hbm.at[didx2], xpr, sem2)

        def process(bufset):
            sd, sidx, didx2, rows, xpr, sem1, sem2, semsc = bufset
            didx = sd.at[1]
            pltpu.make_async_copy(st_hbm.at[sidx], rows, sem1).wait()
            pltpu.make_async_copy(xp_hbm.at[didx2], xpr, sem2).wait()

            @plsc.parallel_loop(0, EC, unroll=8)
            def ebody(e):
                cp = rows[e, pl.ds(VW, 16)]
                z = cp + xpr[e, :]
                a = jnp.maximum(z, NEG_SLOPE * z)
                w = jnp.exp(a)
                rows[e, pl.ds(VW, 16)] = w
                for h in range(HW):
                    wh = _splat16(w, h)
                    rows[e, pl.ds(h * HD, HD)] = wh * rows[e, pl.ds(h * HD, HD)]

            pltpu.async_copy(rows, acc_sh.at[didx], semsc, add=True)

        issue(sid, buf_a, None)

        def guarded(k_, _):
            ci = sid + k_ * NS
            nxt = ci + NS

            @pl.when(ci < NCHUNK)
            def _():
                @pl.when(k_ % 2 == 0)
                def _():
                    @pl.when(nxt < NCHUNK)
                    def _():
                        issue(nxt, buf_b, k_ >= 1)
                    process(buf_a)

                @pl.when(k_ % 2 == 1)
                def _():
                    @pl.when(nxt < NCHUNK)
                    def _():
                        issue(nxt, buf_a, k_ >= 1)
                    process(buf_b)
            return 0

        lax.fori_loop(0, KMAX, guarded, 0)
        for bset in (buf_a, buf_b):
            pltpu.make_async_copy(bset[3], acc_sh.at[bset[0].at[1]],
                                  bset[7]).wait()
        _acc_writeback(acc_sh, out_hbm, cid, sid)

    return k(fi, src2, xp2, zeros).reshape(NC, NP, AW)


def _edge2_sc(fi, q2, kv2, attr, zeros):
    mesh = plsc.VectorSubcoreMesh(**_MESH)

    @functools.partial(
        pl.kernel,
        out_type=jax.ShapeDtypeStruct((NC * NP, AW), jnp.float32),
        mesh=mesh,
        compiler_params=_SC_PARAMS,
        scratch_types=[
            pltpu.VMEM_SHARED((NP, AW), jnp.float32),
        ] + 2 * [
            pltpu.VMEM((2, EC), jnp.int32),
            pltpu.VMEM((EC,), jnp.int32),
            pltpu.VMEM((EC,), jnp.int32),
            pltpu.VMEM((EC, VW), jnp.float32),
            pltpu.VMEM((EC, 2 * VW), jnp.float32),
            pltpu.VMEM((EC, 16), jnp.float32),
            pltpu.VMEM((EC, AW), jnp.float32),
            pltpu.SemaphoreType.DMA,
            pltpu.SemaphoreType.DMA,
            pltpu.SemaphoreType.DMA,
            pltpu.SemaphoreType.DMA,
        ],
    )
    def k(fi_hbm, q_hbm, kv_hbm, at_hbm, z_hbm, out_hbm,
          acc_sh, *bufs):
        buf_a, buf_b = bufs[:11], bufs[11:]
        cid = lax.axis_index("c")
        sid = lax.axis_index("s")
        _acc_zero(z_hbm, acc_sh, sid)
        off = cid * N
        lane_iota = lax.iota(jnp.int32, 16)

        def issue(ci, bufset):
            (sd, sidx, didx2, qrows, kvrows, arows, orows,
             sem1, sem2, sem3, semsc) = bufset
            eb = ci * EC
            pltpu.sync_copy(fi_hbm.at[:, pl.ds(eb, EC)], sd)
            _adjust(sidx, sd.at[0], off)
            _adjust(didx2, sd.at[1], off)
            pltpu.async_copy(q_hbm.at[didx2], qrows, sem1)
            pltpu.async_copy(kv_hbm.at[sidx], kvrows, sem2)
            pltpu.async_copy(at_hbm.at[pl.ds(eb, EC)], arows, sem3)

        def process(bufset, drain):
            (sd, sidx, didx2, qrows, kvrows, arows, orows,
             sem1, sem2, sem3, semsc) = bufset
            didx = sd.at[1]

            @pl.when(drain)
            def _():
                pltpu.make_async_copy(orows, acc_sh.at[didx], semsc).wait()
            pltpu.make_async_copy(q_hbm.at[didx2], qrows, sem1).wait()
            pltpu.make_async_copy(kv_hbm.at[sidx], kvrows, sem2).wait()
            pltpu.make_async_copy(at_hbm.at[pl.ds(0, EC)], arows, sem3).wait()

            @plsc.parallel_loop(0, EC, unroll=8)
            def ebody(e):
                attr_v = arows[e, :]
                atn = jnp.zeros((16,), jnp.float32)
                for h in range(HW):
                    pr = qrows[e, pl.ds(h * HD, HD)] \
                        * kvrows[e, pl.ds(h * HD, HD)] * attr_v
                    sh = _splat16(plsc.cumsum(pr), 15)
                    atn = jnp.where(lane_iota == h, sh, atn)
                w = jnp.exp(atn * 0.25)
                orows[e, pl.ds(VW, 16)] = w
                for h in range(HW):
                    wh = _splat16(w, h)
                    orows[e, pl.ds(h * HD, HD)] = \
                        wh * kvrows[e, pl.ds(VW + h * HD, HD)]

            pltpu.async_copy(orows, acc_sh.at[didx], semsc, add=True)

        issue(sid, buf_a)

        def guarded(k_, _):
            ci = sid + k_ * NS
            nxt = ci + NS

            @pl.when(ci < NCHUNK)
            def _():
                @pl.when(k_ % 2 == 0)
                def _():
                    @pl.when(nxt < NCHUNK)
                    def _():
                        issue(nxt, buf_b)
                    process(buf_a, k_ >= 2)

                @pl.when(k_ % 2 == 1)
                def _():
                    @pl.when(nxt < NCHUNK)
                    def _():
                        issue(nxt, buf_a)
                    process(buf_b, k_ >= 2)
            return 0

        lax.fori_loop(0, KMAX, guarded, 0)
        for bset in (buf_a, buf_b):
            pltpu.make_async_copy(bset[6], acc_sh.at[bset[0].at[1]],
                                  bset[10]).wait()
        _acc_writeback(acc_sh, out_hbm, cid, sid)

    return k(fi, q2, kv2, attr, zeros).reshape(NC, NP, AW)


FP = 102400
FC = 128
FW = FP // NW
FNCH = FW // FC


def _fgather_sc(root3, idx_pad):
    mesh = plsc.VectorSubcoreMesh(**_MESH)

    @functools.partial(
        pl.kernel,
        out_type=jax.ShapeDtypeStruct((FP, ENC), jnp.float32),
        mesh=mesh,
        compiler_params=_SC_PARAMS,
        scratch_types=2 * [
            pltpu.VMEM((FC,), jnp.int32),
            pltpu.VMEM((FC, ENC), jnp.float32),
            pltpu.SemaphoreType.DMA,
            pltpu.SemaphoreType.DMA,
        ],
    )
    def k(t_hbm, i_hbm, out_hbm, *bufs):
        buf_a, buf_b = bufs[:4], bufs[4:]
        cid = lax.axis_index("c")
        sid = lax.axis_index("s")
        wid = sid * NC + cid
        fbase = wid * FW

        def issue(b, bufset, drain):
            iv, rv, sg, so = bufset

            @pl.when(drain)
            def _():
                pltpu.make_async_copy(rv, out_hbm.at[pl.ds(0, FC)], so).wait()
            pltpu.sync_copy(i_hbm.at[pl.ds(b, FC)], iv)
            pltpu.async_copy(t_hbm.at[iv], rv, sg)

        def process(b, bufset):
            iv, rv, sg, so = bufset
            pltpu.make_async_copy(t_hbm.at[iv], rv, sg).wait()
            pltpu.async_copy(rv, out_hbm.at[pl.ds(b, FC)], so)

        issue(fbase, buf_a, jnp.bool_(False))

        def chunk(ci, _):
            b = fbase + ci * FC

            @pl.when(ci % 2 == 0)
            def _():
                @pl.when(ci + 1 < FNCH)
                def _():
                    issue(b + FC, buf_b, ci >= 1)
                process(b, buf_a)

            @pl.when(ci % 2 == 1)
            def _():
                @pl.when(ci + 1 < FNCH)
                def _():
                    issue(b + FC, buf_a, ci >= 1)
                process(b, buf_b)
            return 0

        lax.fori_loop(0, FNCH, chunk, 0)
        pltpu.make_async_copy(buf_a[1], out_hbm.at[pl.ds(0, FC)],
                              buf_a[3]).wait()
        pltpu.make_async_copy(buf_b[1], out_hbm.at[pl.ds(0, FC)],
                              buf_b[3]).wait()

    return k(root3, idx_pad)


def kernel(root_features, feedback_features, feedback_index, fringe_features,
           root_to_fringe_index, root_edge_index, root_edge_attr, W_attn,
           W_ctx2x, W_x2g, W_qkv, W_gate, W_up, W_down, W_fgate, b_fgate,
           W_r2f, b_r2f, rms_node_w, rms_root_w, rms_ffn_w):
    p = jnp.kron(jnp.eye(CH, dtype=jnp.float32), jnp.ones((1, HD), jnp.float32))
    wqkv4 = W_qkv.reshape(ENC, SH, HD, 3)
    wq = wqkv4[..., 0].reshape(ENC, ENC)
    wkv = jnp.concatenate(
        [wqkv4[..., 1].reshape(ENC, ENC), wqkv4[..., 2].reshape(ENC, ENC)],
        axis=1)
    hff = W_gate.shape[1]
    wg = jnp.pad(W_gate, ((0, 0), (0, HFFP - hff)))
    wu = jnp.pad(W_up, ((0, 0), (0, HFFP - hff)))
    wd = jnp.pad(W_down, ((0, HFFP - hff), (0, 0)))

    fi1 = feedback_index.astype(jnp.int32)
    fi2 = root_edge_index.astype(jnp.int32)

    zeros = jnp.zeros((NP, AW), jnp.float32)
    idx_pad = jnp.pad(root_to_fringe_index.astype(jnp.int32), (0, FP - F))

    src2, xp2 = _pre1(feedback_features, root_features, W_attn, W_ctx2x)
    acc1 = _edge1_sc(fi1, src2.reshape(NC * N, AW),
                     xp2.reshape(NC * N, 16), zeros)
    root1, q2, kv2 = _mid(acc1, root_features, W_x2g, rms_node_w, wq, wkv, p)
    acc2 = _edge2_sc(fi2, q2.reshape(NC * N, VW),
                     kv2.reshape(NC * N, 2 * VW), root_edge_attr, zeros)
    fg = _fgate(fringe_features, W_fgate, b_fgate)
    root3, r2f_all = _post(acc2, root1, rms_root_w, wg, wu, wd, rms_ffn_w, p,
                           W_r2f, b_r2f)
    gathered = _fgather_sc(r2f_all, idx_pad)
    fringe_out = _fringe(gathered, fg)
    return (root3, fringe_out)
